# Initial kernel scaffold; baseline (speedup 1.0000x reference)
import jax
import jax.numpy as jnp
from jax.experimental import pallas as pl


def kernel(x, edge_index, edge_feature, W1, b1, W2, b2, gamma, beta):
    raise NotImplementedError("write your pallas kernel here")


# speedup vs baseline: 11.2229x; 11.2229x over previous
import functools

import numpy as np
import jax
import jax.numpy as jnp
from jax import lax
from jax.experimental import pallas as pl
from jax.experimental.pallas import tpu as pltpu
from jax.experimental.pallas import tpu_sc as plsc

N = 10000
E = 320000
D = 128
DE = 16

NP = 10240
C = 128
NCH = 2560
EP = NCH * C
NW = 32
CH_W = NCH // NW
CH_S = NCH // 16
NPS = NP // 16

_BMAT = np.zeros((D, 8), np.float32)
for _j in range(D):
    _BMAT[_j, _j // 16] = 1.0 / 16.0


def _rsqrt16(v):
    u = lax.bitcast_convert_type(v, jnp.int32)
    u = jnp.int32(0x5F3759DF) - (u >> 1)
    y = lax.bitcast_convert_type(u, jnp.float32)
    for _ in range(3):
        y = y * (1.5 - 0.5 * v * y * y)
    return y


def _zero_buf(buf):
    def body(i, _):
        for k in range(8):
            buf[i, pl.ds(k * 16, 16)] = jnp.zeros((16,), jnp.float32)
        return 0
    lax.fori_loop(0, C, body, 0)


def _scale_rows(buf0, w_v, t):
    def scale(g, _):
        wv = w_v[t, pl.ds(g * 16, 16)]
        for l in range(16):
            e = g * 16 + l
            sv = wv[l]
            for k in range(8):
                sl = pl.ds(k * 16, 16)
                buf0[e, sl] = buf0[e, sl] * sv
        return 0
    lax.fori_loop(0, C // 16, scale, 0)


def _agg_group(rowD, colD, wD, h_hbm, acc_sh, buf0, sem):
    for t in range(8):
        pltpu.async_copy(h_hbm.at[rowD.at[t]], buf0, sem).wait()
        _scale_rows(buf0, wD, t)
        pltpu.sync_copy(buf0, acc_sh.at[colD.at[t]], add=True)


def _acc_writeout(acc_sh, part, core, nbase):
    for t in range(NPS // C):
        sl = pl.ds(nbase + t * C, C)
        pltpu.sync_copy(acc_sh.at[sl], part.at[core, sl])


def _sc_conv1_body(rowr, colr, ewr, h1, part, normr, disr,
                   acc_sh, deg_sh, dis_sh,
                   dis_v, buf0, rowD, colD, ewD, nv, sem):
    c = lax.axis_index("c")
    s = lax.axis_index("s")
    base = s * CH_S + c * CH_W
    nbase = s * NPS

    _zero_buf(buf0)
    for t in range(NPS // C):
        pltpu.sync_copy(buf0, acc_sh.at[pl.ds(nbase + t * C, C)])

    def ones_body(i, _):
        nv[pl.ds(i * 16, 16)] = jnp.ones((16,), jnp.float32)
        return 0
    lax.fori_loop(0, NPS // 16, ones_body, 0)
    pltpu.sync_copy(nv, deg_sh.at[pl.ds(nbase, NPS)])

    plsc.subcore_barrier()

    dbase = s * CH_S

    def deg_body(g, _):
        pltpu.sync_copy(colr.at[pl.ds(dbase + g * 8, 8)], colD)
        pltpu.sync_copy(ewr.at[pl.ds(dbase + g * 8, 8)], ewD)
        for t in range(8):
            pltpu.sync_copy(ewD.at[t], deg_sh.at[colD.at[t]], add=True)
        return 0
    lax.fori_loop(0, CH_S // 8, deg_body, 0)

    plsc.subcore_barrier()

    pltpu.sync_copy(deg_sh.at[pl.ds(nbase, NPS)], nv)

    def dis_body(i, _):
        sl = pl.ds(i * 16, 16)
        nv[sl] = _rsqrt16(nv[sl])
        return 0
    lax.fori_loop(0, NPS // 16, dis_body, 0)
    pltpu.sync_copy(nv, dis_sh.at[pl.ds(nbase, NPS)])

    @pl.when(c == 0)
    def _():
        pltpu.sync_copy(nv, disr.at[pl.ds(nbase, NPS)])

    plsc.subcore_barrier()

    pltpu.sync_copy(dis_sh, dis_v)

    def group(g, _):
        gb = base + g * 8
        pltpu.sync_copy(rowr.at[pl.ds(gb, 8)], rowD)
        pltpu.sync_copy(colr.at[pl.ds(gb, 8)], colD)
        pltpu.sync_copy(ewr.at[pl.ds(gb, 8)], ewD)
        for t in range(8):
            for k in range(8):
                sl = pl.ds(k * 16, 16)
                dr = plsc.load_gather(dis_v, [rowD[t, sl]])
                dq = plsc.load_gather(dis_v, [colD[t, sl]])
                ewD[t, sl] = dr * ewD[t, sl] * dq
        pltpu.sync_copy(ewD, normr.at[pl.ds(gb, 8)])
        _agg_group(rowD, colD, ewD, h1, acc_sh, buf0, sem)
        return 0
    lax.fori_loop(0, CH_W // 8, group, 0)

    plsc.subcore_barrier()
    _acc_writeout(acc_sh, part, c, nbase)


def _sc_conv2_body(rowr, colr, normr, h2, part,
                   acc_sh, buf0, rowD, colD, wD, sem):
    c = lax.axis_index("c")
    s = lax.axis_index("s")
    base = s * CH_S + c * CH_W
    nbase = s * NPS

    _zero_buf(buf0)
    for t in range(NPS // C):
        pltpu.sync_copy(buf0, acc_sh.at[pl.ds(nbase + t * C, C)])

    plsc.subcore_barrier()

    def group(g, _):
        gb = base + g * 8
        pltpu.sync_copy(rowr.at[pl.ds(gb, 8)], rowD)
        pltpu.sync_copy(colr.at[pl.ds(gb, 8)], colD)
        pltpu.sync_copy(normr.at[pl.ds(gb, 8)], wD)
        _agg_group(rowD, colD, wD, h2, acc_sh, buf0, sem)
        return 0
    lax.fori_loop(0, CH_W // 8, group, 0)

    plsc.subcore_barrier()
    _acc_writeout(acc_sh, part, c, nbase)


RB = 1000
EB = 4096


def _mm(a, b):
    return jnp.dot(a, b, preferred_element_type=jnp.float32,
                   precision=lax.Precision.HIGHEST)


def _tc_h1_body(x_ref, w1_ref, h1_ref):
    h1_ref[...] = _mm(x_ref[...], w1_ref[...])


def _tc_ew_body(efr_ref, bmat_ref, ewr_ref):
    ewr_ref[...] = _mm(efr_ref[...], bmat_ref[...])


def _tc_mid1_body(p0_ref, p1_ref, h1_ref, dis_ref, b1_ref, a_ref, st_ref):
    i = pl.program_id(0)
    inv = dis_ref[...] * dis_ref[...]
    a = (p0_ref[...] + p1_ref[...] + h1_ref[...] * inv
         + b1_ref[...][None, :])
    a_ref[...] = a

    @pl.when(i == 0)
    def _():
        st_ref[...] = jnp.zeros_like(st_ref)
    st_ref[0, :] += jnp.sum(a, axis=0)
    st_ref[1, :] += jnp.sum(a * a, axis=0)


def _tc_mid2_body(a_ref, st_ref, g_ref, bt_ref, w2_ref, h2_ref):
    mean = st_ref[0, :] * (1.0 / N)
    var = st_ref[1, :] * (1.0 / N) - mean * mean
    scale = g_ref[...] / jnp.sqrt(var + 1e-5)
    shift = bt_ref[...] - mean * scale
    hb = a_ref[...] * scale[None, :] + shift[None, :]
    hb = jnp.maximum(hb, 0.0)
    h2_ref[...] = _mm(hb, w2_ref[...])


def _tc_fin_body(p0_ref, p1_ref, h2_ref, dis_ref, b2_ref, out_ref):
    inv = dis_ref[...] * dis_ref[...]
    out_ref[...] = (p0_ref[...] + p1_ref[...] + h2_ref[...] * inv
                    + b2_ref[...][None, :])


def kernel(x, edge_index, edge_feature, W1, b1, W2, b2, gamma, beta):
    row = edge_index[0]
    col = edge_index[1]
    npad = EP - E
    padi = (jnp.arange(npad, dtype=jnp.int32) * 797) % N
    rowp = jnp.concatenate([row, padi]).reshape(NCH, C)
    colp = jnp.concatenate([col, padi]).reshape(NCH, C)
    efp = jnp.concatenate(
        [edge_feature, jnp.zeros((npad, DE), jnp.float32)]).reshape(EP // 8, D)
    bmat = jnp.asarray(_BMAT)

    h1 = pl.pallas_call(
        _tc_h1_body,
        grid=(N // RB,),
        in_specs=[
            pl.BlockSpec((RB, D), lambda i: (i, 0)),
            pl.BlockSpec((D, D), lambda i: (0, 0)),
        ],
        out_specs=pl.BlockSpec((RB, D), lambda i: (i, 0)),
        out_shape=jax.ShapeDtypeStruct((N, D), jnp.float32),
    )(x, W1)
    ewr8 = pl.pallas_call(
        _tc_ew_body,
        grid=(EP // 8 // EB,),
        in_specs=[
            pl.BlockSpec((EB, D), lambda i: (i, 0)),
            pl.BlockSpec((D, 8), lambda i: (0, 0)),
        ],
        out_specs=pl.BlockSpec((EB, 8), lambda i: (i, 0)),
        out_shape=jax.ShapeDtypeStruct((EP // 8, 8), jnp.float32),
    )(efp, bmat)
    ewr = ewr8.reshape(NCH, C)

    mesh = plsc.VectorSubcoreMesh(core_axis_name="c", subcore_axis_name="s")

    sc_conv1 = functools.partial(
        pl.kernel,
        mesh=mesh,
        compiler_params=pltpu.CompilerParams(needs_layout_passes=False),
        out_type=[
            jax.ShapeDtypeStruct((2, NP, D), jnp.float32),
            jax.ShapeDtypeStruct((NCH, C), jnp.float32),
            jax.ShapeDtypeStruct((NP,), jnp.float32),
        ],
        scratch_types=[
            pltpu.VMEM_SHARED((NP, D), jnp.float32),
            pltpu.VMEM_SHARED((NP,), jnp.float32),
            pltpu.VMEM_SHARED((NP,), jnp.float32),
            pltpu.VMEM((NP,), jnp.float32),
            pltpu.VMEM((C, D), jnp.float32),
            pltpu.VMEM((8, C), jnp.int32),
            pltpu.VMEM((8, C), jnp.int32),
            pltpu.VMEM((8, C), jnp.float32),
            pltpu.VMEM((NPS,), jnp.float32),
            pltpu.SemaphoreType.DMA,
        ],
    )(_sc_conv1_body)
    part1, normr, dis = sc_conv1(rowp, colp, ewr, h1)

    p10 = part1[0]
    p11 = part1[1]
    dis2 = dis.reshape(NP, 1)
    a1, st = pl.pallas_call(
        _tc_mid1_body,
        grid=(N // RB,),
        in_specs=[
            pl.BlockSpec((RB, D), lambda i: (i, 0)),
            pl.BlockSpec((RB, D), lambda i: (i, 0)),
            pl.BlockSpec((RB, D), lambda i: (i, 0)),
            pl.BlockSpec((RB, 1), lambda i: (i, 0)),
            pl.BlockSpec((D,), lambda i: (0,)),
        ],
        out_specs=[
            pl.BlockSpec((RB, D), lambda i: (i, 0)),
            pl.BlockSpec((2, D), lambda i: (0, 0)),
        ],
        out_shape=[
            jax.ShapeDtypeStruct((N, D), jnp.float32),
            jax.ShapeDtypeStruct((2, D), jnp.float32),
        ],
    )(p10, p11, h1, dis2, b1)
    h2 = pl.pallas_call(
        _tc_mid2_body,
        grid=(N // RB,),
        in_specs=[
            pl.BlockSpec((RB, D), lambda i: (i, 0)),
            pl.BlockSpec((2, D), lambda i: (0, 0)),
            pl.BlockSpec((D,), lambda i: (0,)),
            pl.BlockSpec((D,), lambda i: (0,)),
            pl.BlockSpec((D, D), lambda i: (0, 0)),
        ],
        out_specs=pl.BlockSpec((RB, D), lambda i: (i, 0)),
        out_shape=jax.ShapeDtypeStruct((N, D), jnp.float32),
    )(a1, st, gamma, beta, W2)

    sc_conv2 = functools.partial(
        pl.kernel,
        mesh=mesh,
        compiler_params=pltpu.CompilerParams(needs_layout_passes=False),
        out_type=jax.ShapeDtypeStruct((2, NP, D), jnp.float32),
        scratch_types=[
            pltpu.VMEM_SHARED((NP, D), jnp.float32),
            pltpu.VMEM((C, D), jnp.float32),
            pltpu.VMEM((8, C), jnp.int32),
            pltpu.VMEM((8, C), jnp.int32),
            pltpu.VMEM((8, C), jnp.float32),
            pltpu.SemaphoreType.DMA,
        ],
    )(_sc_conv2_body)
    part2 = sc_conv2(rowp, colp, normr, h2)

    out = pl.pallas_call(
        _tc_fin_body,
        grid=(N // RB,),
        in_specs=[
            pl.BlockSpec((RB, D), lambda i: (i, 0)),
            pl.BlockSpec((RB, D), lambda i: (i, 0)),
            pl.BlockSpec((RB, D), lambda i: (i, 0)),
            pl.BlockSpec((RB, 1), lambda i: (i, 0)),
            pl.BlockSpec((D,), lambda i: (0,)),
        ],
        out_specs=pl.BlockSpec((RB, D), lambda i: (i, 0)),
        out_shape=jax.ShapeDtypeStruct((N, D), jnp.float32),
    )(part2[0], part2[1], h2, dis2, b2)
    return out


# speedup vs baseline: 13.3367x; 1.1884x over previous
import functools

import numpy as np
import jax
import jax.numpy as jnp
from jax import lax
from jax.experimental import pallas as pl
from jax.experimental.pallas import tpu as pltpu
from jax.experimental.pallas import tpu_sc as plsc

N = 10000
E = 320000
D = 128
DE = 16

NP = 10240
C = 128
NCH = 2560
EP = NCH * C
NW = 32
CH_W = NCH // NW
CH_S = NCH // 16
NPS = NP // 16

RB = 1000
EB = 4096

_BMAT = np.zeros((D, 8), np.float32)
for _j in range(D):
    _BMAT[_j, _j // 16] = 1.0 / 16.0


def _rsqrt16(v):
    u = lax.bitcast_convert_type(v, jnp.int32)
    u = jnp.int32(0x5F3759DF) - (u >> 1)
    y = lax.bitcast_convert_type(u, jnp.float32)
    for _ in range(3):
        y = y * (1.5 - 0.5 * v * y * y)
    return y


def _sc_norm_body(rowr, colr, ewr, normr, disr,
                  deg_sh, dis_sh, dis_v, rowD, colD, ewD, nv):
    c = lax.axis_index("c")
    s = lax.axis_index("s")
    base = s * CH_S + c * CH_W
    nbase = s * NPS

    def ones_body(i, _):
        nv[pl.ds(i * 16, 16)] = jnp.ones((16,), jnp.float32)
        return 0
    lax.fori_loop(0, NPS // 16, ones_body, 0)
    pltpu.sync_copy(nv, deg_sh.at[pl.ds(nbase, NPS)])

    plsc.subcore_barrier()

    dbase = s * CH_S

    def deg_body(g, _):
        pltpu.sync_copy(colr.at[pl.ds(dbase + g * 8, 8)], colD)
        pltpu.sync_copy(ewr.at[pl.ds(dbase + g * 8, 8)], ewD)
        for t in range(8):
            pltpu.sync_copy(ewD.at[t], deg_sh.at[colD.at[t]], add=True)
        return 0
    lax.fori_loop(0, CH_S // 8, deg_body, 0)

    plsc.subcore_barrier()

    pltpu.sync_copy(deg_sh.at[pl.ds(nbase, NPS)], nv)

    def dis_body(i, _):
        sl = pl.ds(i * 16, 16)
        nv[sl] = _rsqrt16(nv[sl])
        return 0
    lax.fori_loop(0, NPS // 16, dis_body, 0)
    pltpu.sync_copy(nv, dis_sh.at[pl.ds(nbase, NPS)])

    @pl.when(c == 0)
    def _():
        pltpu.sync_copy(nv, disr.at[pl.ds(nbase, NPS)])

    plsc.subcore_barrier()

    pltpu.sync_copy(dis_sh, dis_v)

    def group(g, _):
        gb = base + g * 8
        pltpu.sync_copy(rowr.at[pl.ds(gb, 8)], rowD)
        pltpu.sync_copy(colr.at[pl.ds(gb, 8)], colD)
        pltpu.sync_copy(ewr.at[pl.ds(gb, 8)], ewD)
        for t in range(8):
            for k in range(8):
                sl = pl.ds(k * 16, 16)
                dr = plsc.load_gather(dis_v, [rowD[t, sl]])
                dq = plsc.load_gather(dis_v, [colD[t, sl]])
                ewD[t, sl] = dr * ewD[t, sl] * dq
        pltpu.sync_copy(ewD, normr.at[pl.ds(gb, 8)])
        return 0
    lax.fori_loop(0, CH_W // 8, group, 0)


def _sc_agg_body(rowr, colr, normr, h_hbm, part,
                 acc_sh, rowD, colD, wD, buf0, buf1,
                 sg0, sg1, ss0, ss1):
    c = lax.axis_index("c")
    s = lax.axis_index("s")
    base = s * CH_S + c * CH_W
    nbase = s * NPS

    bufs = (buf0, buf1)
    sgs = (sg0, sg1)
    sss = (ss0, ss1)

    def zrow(i, _):
        for k in range(D // 16):
            buf0[i, pl.ds(k * 16, 16)] = jnp.zeros((16,), jnp.float32)
        return 0
    lax.fori_loop(0, C, zrow, 0)
    for t in range(NPS // C):
        pltpu.sync_copy(buf0, acc_sh.at[pl.ds(nbase + t * C, C)])

    plsc.subcore_barrier()

    def group(g, _):
        gb = base + g * 8
        pltpu.sync_copy(rowr.at[pl.ds(gb, 8)], rowD)
        pltpu.sync_copy(colr.at[pl.ds(gb, 8)], colD)
        pltpu.sync_copy(normr.at[pl.ds(gb, 8)], wD)

        dg = [None, None]
        dsc = [None, None]
        dg[0] = pltpu.async_copy(h_hbm.at[rowD.at[0]], buf0, sg0)
        for t in range(8):
            b = t % 2
            nb = 1 - b
            if t + 1 < 8:
                if dsc[nb] is not None:
                    dsc[nb].wait()
                dg[nb] = pltpu.async_copy(
                    h_hbm.at[rowD.at[t + 1]], bufs[nb], sgs[nb])
            dg[b].wait()

            def scale(q, _):
                wv = wD[t, pl.ds(q * 16, 16)]
                for l in range(16):
                    e = q * 16 + l
                    sv = wv[l]
                    for k in range(D // 16):
                        sl = pl.ds(k * 16, 16)
                        bufs[b][e, sl] = bufs[b][e, sl] * sv
                return 0
            lax.fori_loop(0, C // 16, scale, 0)

            dsc[b] = pltpu.async_copy(
                bufs[b], acc_sh.at[colD.at[t]], sss[b], add=True)
        dsc[0].wait()
        dsc[1].wait()
        return 0
    lax.fori_loop(0, CH_W // 8, group, 0)

    plsc.subcore_barrier()
    for t in range(NPS // C):
        sl = pl.ds(nbase + t * C, C)
        pltpu.sync_copy(acc_sh.at[sl], part.at[c, sl])


def _mm(a, b):
    return jnp.dot(a, b, preferred_element_type=jnp.float32,
                   precision=lax.Precision.HIGHEST)


def _tc_h1_body(x_ref, w1_ref, h1_ref):
    h1_ref[...] = _mm(x_ref[...], w1_ref[...])


def _tc_ew_body(efr_ref, bmat_ref, ewr_ref):
    ewr_ref[...] = _mm(efr_ref[...], bmat_ref[...])


def _tc_mid1_body(p0_ref, p1_ref, h1_ref, dis_ref, b1_ref, a_ref, st_ref):
    i = pl.program_id(0)
    inv = dis_ref[...] * dis_ref[...]
    a = (p0_ref[...] + p1_ref[...] + h1_ref[...] * inv
         + b1_ref[...][None, :])
    a_ref[...] = a

    @pl.when(i == 0)
    def _():
        st_ref[...] = jnp.zeros_like(st_ref)
    st_ref[0, :] += jnp.sum(a, axis=0)
    st_ref[1, :] += jnp.sum(a * a, axis=0)


def _tc_mid2_body(a_ref, st_ref, g_ref, bt_ref, w2_ref, h2_ref):
    mean = st_ref[0, :] * (1.0 / N)
    var = st_ref[1, :] * (1.0 / N) - mean * mean
    scale = g_ref[...] / jnp.sqrt(var + 1e-5)
    shift = bt_ref[...] - mean * scale
    hb = a_ref[...] * scale[None, :] + shift[None, :]
    hb = jnp.maximum(hb, 0.0)
    h2_ref[...] = _mm(hb, w2_ref[...])


def _tc_fin_body(p0_ref, p1_ref, h2_ref, dis_ref, b2_ref, out_ref):
    inv = dis_ref[...] * dis_ref[...]
    out_ref[...] = (p0_ref[...] + p1_ref[...] + h2_ref[...] * inv
                    + b2_ref[...][None, :])


def kernel(x, edge_index, edge_feature, W1, b1, W2, b2, gamma, beta):
    row = edge_index[0]
    col = edge_index[1]
    npad = EP - E
    padi = (jnp.arange(npad, dtype=jnp.int32) * 797) % N
    rowp = jnp.concatenate([row, padi]).reshape(NCH, C)
    colp = jnp.concatenate([col, padi]).reshape(NCH, C)
    efp = jnp.concatenate(
        [edge_feature, jnp.zeros((npad, DE), jnp.float32)]).reshape(EP // 8, D)
    bmat = jnp.asarray(_BMAT)

    h1 = pl.pallas_call(
        _tc_h1_body,
        grid=(N // RB,),
        in_specs=[
            pl.BlockSpec((RB, D), lambda i: (i, 0)),
            pl.BlockSpec((D, D), lambda i: (0, 0)),
        ],
        out_specs=pl.BlockSpec((RB, D), lambda i: (i, 0)),
        out_shape=jax.ShapeDtypeStruct((N, D), jnp.float32),
    )(x, W1)
    ewr8 = pl.pallas_call(
        _tc_ew_body,
        grid=(EP // 8 // EB,),
        in_specs=[
            pl.BlockSpec((EB, D), lambda i: (i, 0)),
            pl.BlockSpec((D, 8), lambda i: (0, 0)),
        ],
        out_specs=pl.BlockSpec((EB, 8), lambda i: (i, 0)),
        out_shape=jax.ShapeDtypeStruct((EP // 8, 8), jnp.float32),
    )(efp, bmat)
    ewr = ewr8.reshape(NCH, C)

    mesh = plsc.VectorSubcoreMesh(core_axis_name="c", subcore_axis_name="s")
    scp = pltpu.CompilerParams(needs_layout_passes=False)

    sc_norm = functools.partial(
        pl.kernel,
        mesh=mesh,
        compiler_params=scp,
        out_type=[
            jax.ShapeDtypeStruct((NCH, C), jnp.float32),
            jax.ShapeDtypeStruct((NP,), jnp.float32),
        ],
        scratch_types=[
            pltpu.VMEM_SHARED((NP,), jnp.float32),
            pltpu.VMEM_SHARED((NP,), jnp.float32),
            pltpu.VMEM((NP,), jnp.float32),
            pltpu.VMEM((8, C), jnp.int32),
            pltpu.VMEM((8, C), jnp.int32),
            pltpu.VMEM((8, C), jnp.float32),
            pltpu.VMEM((NPS,), jnp.float32),
        ],
    )(_sc_norm_body)
    normr, dis = sc_norm(rowp, colp, ewr)

    sc_agg = functools.partial(
        pl.kernel,
        mesh=mesh,
        compiler_params=scp,
        out_type=jax.ShapeDtypeStruct((2, NP, D), jnp.float32),
        scratch_types=[
            pltpu.VMEM_SHARED((NP, D), jnp.float32),
            pltpu.VMEM((8, C), jnp.int32),
            pltpu.VMEM((8, C), jnp.int32),
            pltpu.VMEM((8, C), jnp.float32),
            pltpu.VMEM((C, D), jnp.float32),
            pltpu.VMEM((C, D), jnp.float32),
            pltpu.SemaphoreType.DMA,
            pltpu.SemaphoreType.DMA,
            pltpu.SemaphoreType.DMA,
            pltpu.SemaphoreType.DMA,
        ],
    )(_sc_agg_body)

    part1 = sc_agg(rowp, colp, normr, h1)

    dis2 = dis.reshape(NP, 1)
    a1, st = pl.pallas_call(
        _tc_mid1_body,
        grid=(N // RB,),
        in_specs=[
            pl.BlockSpec((RB, D), lambda i: (i, 0)),
            pl.BlockSpec((RB, D), lambda i: (i, 0)),
            pl.BlockSpec((RB, D), lambda i: (i, 0)),
            pl.BlockSpec((RB, 1), lambda i: (i, 0)),
            pl.BlockSpec((D,), lambda i: (0,)),
        ],
        out_specs=[
            pl.BlockSpec((RB, D), lambda i: (i, 0)),
            pl.BlockSpec((2, D), lambda i: (0, 0)),
        ],
        out_shape=[
            jax.ShapeDtypeStruct((N, D), jnp.float32),
            jax.ShapeDtypeStruct((2, D), jnp.float32),
        ],
    )(part1[0], part1[1], h1, dis2, b1)

    h2 = pl.pallas_call(
        _tc_mid2_body,
        grid=(N // RB,),
        in_specs=[
            pl.BlockSpec((RB, D), lambda i: (i, 0)),
            pl.BlockSpec((2, D), lambda i: (0, 0)),
            pl.BlockSpec((D,), lambda i: (0,)),
            pl.BlockSpec((D,), lambda i: (0,)),
            pl.BlockSpec((D, D), lambda i: (0, 0)),
        ],
        out_specs=pl.BlockSpec((RB, D), lambda i: (i, 0)),
        out_shape=jax.ShapeDtypeStruct((N, D), jnp.float32),
    )(a1, st, gamma, beta, W2)

    part2 = sc_agg(rowp, colp, normr, h2)

    out = pl.pallas_call(
        _tc_fin_body,
        grid=(N // RB,),
        in_specs=[
            pl.BlockSpec((RB, D), lambda i: (i, 0)),
            pl.BlockSpec((RB, D), lambda i: (i, 0)),
            pl.BlockSpec((RB, D), lambda i: (i, 0)),
            pl.BlockSpec((RB, 1), lambda i: (i, 0)),
            pl.BlockSpec((D,), lambda i: (0,)),
        ],
        out_specs=pl.BlockSpec((RB, D), lambda i: (i, 0)),
        out_shape=jax.ShapeDtypeStruct((N, D), jnp.float32),
    )(part2[0], part2[1], h2, dis2, b2)
    return out
